# Initial kernel scaffold; baseline (speedup 1.0000x reference)
import functools

import jax
import jax.numpy as jnp
from jax import lax
from jax.experimental import pallas as pl
from jax.experimental.pallas import tpu as pltpu
from jax.experimental.pallas import tpu_sc as plsc

N = 10000
D = 128
E = 320000
NS = 16
EP = E // NS
CH = 80
NCHUNK = EP // CH
RP = N // NS
RC = 125
NRC = RP // RC


def _step_body(table, gidx_h, sidx_h, sw_h, out,
               accum, degs,
               gidx, sidx, rows, ones_v, deg_v, acc_v, old_v, sw_v):
    c = lax.axis_index("c")
    t = lax.axis_index("s")
    node0 = t * RP
    obase = (1 - c) * N + node0

    pltpu.sync_copy(gidx_h.at[c, t], gidx)
    pltpu.sync_copy(sidx_h.at[c, t], sidx)
    pltpu.sync_copy(sw_h.at[c], sw_v)

    zeros16 = jnp.zeros((16,), jnp.float32)
    ones16 = jnp.ones((16,), jnp.float32)

    def zacc(r, carry):
        for j in range(8):
            acc_v[r, pl.ds(16 * j, 16)] = zeros16
        return carry
    lax.fori_loop(0, RC, zacc, 0)
    for cc in range(NRC):
        pltpu.sync_copy(acc_v, accum.at[pl.ds(node0 + cc * RC, RC)])

    def zdeg(r, carry):
        deg_v[r] = zeros16
        return carry
    lax.fori_loop(0, RP, zdeg, 0)
    pltpu.sync_copy(deg_v, degs.at[pl.ds(node0, RP)])

    def fones(r, carry):
        ones_v[r] = ones16
        return carry
    lax.fori_loop(0, CH, fones, 0)

    plsc.subcore_barrier()

    def edge(j, carry):
        pltpu.sync_copy(table.at[gidx.at[j]], rows)
        pltpu.sync_copy(ones_v, degs.at[sidx.at[j]], add=True)
        pltpu.sync_copy(rows, accum.at[sidx.at[j]], add=True)
        return carry
    lax.fori_loop(0, NCHUNK, edge, 0)

    plsc.subcore_barrier()

    pltpu.sync_copy(degs.at[pl.ds(node0, RP)], deg_v)
    sw = sw_v[...]
    for cc in range(NRC):
        nb = node0 + cc * RC
        ob = obase + cc * RC
        pltpu.sync_copy(accum.at[pl.ds(nb, RC)], acc_v)
        pltpu.sync_copy(table.at[pl.ds(ob, RC)], old_v)

        def rowf(r, carry):
            dv = deg_v[cc * RC + r]
            rv = 1.0 / jnp.maximum(dv, 1.0)
            for j in range(8):
                sl = pl.ds(16 * j, 16)
                acc_v[r, sl] = acc_v[r, sl] * rv + old_v[r, sl] * sw
            return carry
        lax.fori_loop(0, RC, rowf, 0)
        pltpu.sync_copy(acc_v, out.at[pl.ds(ob, RC)])


_step = functools.partial(
    pl.kernel,
    out_type=jax.ShapeDtypeStruct((2 * N, D), jnp.float32),
    mesh=plsc.VectorSubcoreMesh(core_axis_name="c", subcore_axis_name="s"),
    scratch_types=[
        pltpu.VMEM_SHARED((N, D), jnp.float32),
        pltpu.VMEM_SHARED((N, 16), jnp.float32),
        pltpu.VMEM((NCHUNK, CH), jnp.int32),
        pltpu.VMEM((NCHUNK, CH), jnp.int32),
        pltpu.VMEM((CH, D), jnp.float32),
        pltpu.VMEM((CH, 16), jnp.float32),
        pltpu.VMEM((RP, 16), jnp.float32),
        pltpu.VMEM((RC, D), jnp.float32),
        pltpu.VMEM((RC, D), jnp.float32),
        pltpu.VMEM((16,), jnp.float32),
    ],
)(_step_body)


def kernel(edge_index, user_emb, item_emb, u_sw, i_sw):
    src = edge_index[0].reshape(NS, NCHUNK, CH)
    dst = edge_index[1].reshape(NS, NCHUNK, CH)
    gidx_all = jnp.stack([src, dst + N])
    sidx_all = jnp.stack([dst, src])
    sw_all = jnp.stack([jnp.broadcast_to(i_sw, (16,)),
                        jnp.broadcast_to(u_sw, (16,))])
    table = jnp.concatenate([user_emb, item_emb], axis=0)
    table = _step(table, gidx_all, sidx_all, sw_all)
    table = _step(table, gidx_all, sidx_all, sw_all)
    return table[:N], table[N:]


# speedup vs baseline: 4.8651x; 4.8651x over previous
import functools

import jax
import jax.numpy as jnp
from jax import lax
from jax.experimental import pallas as pl
from jax.experimental.pallas import tpu as pltpu
from jax.experimental.pallas import tpu_sc as plsc

N = 10000
NP = 10240
D = 128
E = 320000
NS = 16
EP = E // NS
CH = 80
NCHUNK = EP // CH
RP = NP // NS
RC = 64
NRC = RP // RC


def _step_body(table, gidx_h, sidx_h, sw_h, out,
               accum, degs,
               gidx_v, sidx_v, rows, ones_v, deg_v, acc_v, old_v, sw_v):
    c = lax.axis_index("c")
    t = lax.axis_index("s")
    node0 = t * RP
    obase = (1 - c) * NP + node0
    ebase = (c * NS + t) * EP

    pltpu.sync_copy(sw_h.at[c], sw_v)

    zeros16 = jnp.zeros((16,), jnp.float32)
    ones16 = jnp.ones((16,), jnp.float32)

    def zacc(r, carry):
        for j in range(8):
            acc_v[r, pl.ds(16 * j, 16)] = zeros16
        return carry
    lax.fori_loop(0, RC, zacc, 0)
    for cc in range(NRC):
        pltpu.sync_copy(acc_v, accum.at[pl.ds(node0 + cc * RC, RC)])

    def zdeg(r, carry):
        deg_v[r] = zeros16
        return carry
    lax.fori_loop(0, RC, zdeg, 0)
    for cc in range(NRC):
        pltpu.sync_copy(deg_v, degs.at[pl.ds(node0 + cc * RC, RC)])

    def fones(r, carry):
        ones_v[r] = ones16
        return carry
    lax.fori_loop(0, CH, fones, 0)

    plsc.subcore_barrier()

    def edge(j, carry):
        pltpu.sync_copy(gidx_h.at[pl.ds(ebase + j * CH, CH)], gidx_v)
        pltpu.sync_copy(sidx_h.at[pl.ds(ebase + j * CH, CH)], sidx_v)
        pltpu.sync_copy(table.at[gidx_v], rows)
        pltpu.sync_copy(ones_v, degs.at[sidx_v], add=True)
        pltpu.sync_copy(rows, accum.at[sidx_v], add=True)
        return carry
    lax.fori_loop(0, NCHUNK, edge, 0)

    plsc.subcore_barrier()

    sw = sw_v[0]
    for cc in range(NRC):
        nb = node0 + cc * RC
        ob = obase + cc * RC
        pltpu.sync_copy(degs.at[pl.ds(nb, RC)], deg_v)
        pltpu.sync_copy(accum.at[pl.ds(nb, RC)], acc_v)
        pltpu.sync_copy(table.at[pl.ds(ob, RC)], old_v)

        def rowf(r, carry):
            dv = deg_v[r]
            rv = 1.0 / jnp.maximum(dv, 1.0)
            for j in range(8):
                sl = pl.ds(16 * j, 16)
                acc_v[r, sl] = acc_v[r, sl] * rv + old_v[r, sl] * sw
            return carry
        lax.fori_loop(0, RC, rowf, 0)
        pltpu.sync_copy(acc_v, out.at[pl.ds(ob, RC)])


_step = functools.partial(
    pl.kernel,
    out_type=jax.ShapeDtypeStruct((2 * NP, D), jnp.float32),
    mesh=plsc.VectorSubcoreMesh(core_axis_name="c", subcore_axis_name="s"),
    compiler_params=pltpu.CompilerParams(use_tc_tiling_on_sc=False),
    scratch_types=[
        pltpu.VMEM_SHARED((NP, D), jnp.float32),
        pltpu.VMEM_SHARED((NP, 16), jnp.float32),
        pltpu.VMEM((CH,), jnp.int32),
        pltpu.VMEM((CH,), jnp.int32),
        pltpu.VMEM((CH, D), jnp.float32),
        pltpu.VMEM((CH, 16), jnp.float32),
        pltpu.VMEM((RC, 16), jnp.float32),
        pltpu.VMEM((RC, D), jnp.float32),
        pltpu.VMEM((RC, D), jnp.float32),
        pltpu.VMEM((8, 16), jnp.float32),
    ],
)(_step_body)


def kernel(edge_index, user_emb, item_emb, u_sw, i_sw):
    src = edge_index[0]
    dst = edge_index[1]
    gidx_all = jnp.concatenate([src, dst + NP])
    sidx_all = jnp.concatenate([dst, src])
    sw_all = jnp.stack([jnp.broadcast_to(i_sw, (8, 16)),
                        jnp.broadcast_to(u_sw, (8, 16))])
    zpad = jnp.zeros((NP - N, D), jnp.float32)
    table = jnp.concatenate([user_emb, zpad, item_emb, zpad], axis=0)
    table = _step(table, gidx_all, sidx_all, sw_all)
    table = _step(table, gidx_all, sidx_all, sw_all)
    return table[:N], table[NP:NP + N]
